# baseline (speedup 1.0000x reference)
import dataclasses

import jax
import jax.numpy as jnp
from jax import lax
from jax.experimental import pallas as pl
from jax.experimental.pallas import tpu as pltpu
from jax.experimental.pallas import tpu_sc as plsc

MAX_CONTEXT = 200
PADDING_TOKEN = 0

_NC = 2
_NS = 16
_NW = _NC * _NS
_GRP = 16

_SC_ROWS = 2048

_BB = 128
_K = 4

_CHUNK_OFFS = tuple(range(0, 192, 16)) + (184,)


def _sc_kernel_call(inputs, emb_flat, batch, seq, row_elems):
    rpw = batch // _NW
    mesh = plsc.VectorSubcoreMesh(core_axis_name="c", subcore_axis_name="s")

    def body(inp_hbm, emb_hbm, out_hbm, emb_v, inp_v, dirty_v, sem):
        c = lax.axis_index("c")
        s = lax.axis_index("s")
        wid = s * _NC + c
        base = wid * rpw
        pltpu.sync_copy(emb_hbm, emb_v)

        def _drain(count):
            def _w(_, x):
                pltpu.make_async_copy(emb_v, out_hbm.at[pl.ds(base, 1)], sem).wait()
                return x

            lax.fori_loop(0, count, _w, 0)

        def _group(g, prev_fired):
            _drain(prev_fired)
            row0 = base + g * _GRP
            pltpu.sync_copy(inp_hbm.at[pl.ds(row0, _GRP)], inp_v)
            fired = jnp.int32(0)
            for r in range(_GRP):
                ok = None
                for off in _CHUNK_OFFS:
                    nz = inp_v[r, pl.ds(off, 16)] != PADDING_TOKEN
                    ok = nz if ok is None else jnp.logical_and(ok, nz)
                clean = jnp.all(ok)
                fired = fired + clean.astype(jnp.int32)

                @pl.when(clean)
                def _fast():
                    pltpu.make_async_copy(
                        emb_v, out_hbm.at[pl.ds(row0 + r, 1)], sem
                    ).start()

                @pl.when(jnp.logical_not(clean))
                def _masked():
                    lane = jnp.arange(16, dtype=jnp.int32)
                    zero16 = jnp.zeros((16,), jnp.int32)

                    @pl.loop(0, row_elems // 16)
                    def _chunk(ch):
                        col = ch * 16
                        tok = col // 64
                        tok16 = jnp.broadcast_to(tok, (16,))
                        r16 = jnp.broadcast_to(jnp.int32(r), (16,))
                        tokval = plsc.load_gather(inp_v, [r16, tok16])
                        m = (tokval != PADDING_TOKEN).astype(jnp.float32)
                        cols = col + lane
                        ev = plsc.load_gather(emb_v, [zero16, cols])
                        plsc.store_scatter(dirty_v, [zero16, cols], ev * m)

                    pltpu.sync_copy(dirty_v, out_hbm.at[pl.ds(row0 + r, 1)])

            return fired

        last = lax.fori_loop(0, rpw // _GRP, _group, jnp.int32(0))
        _drain(last)

    cp = pltpu.CompilerParams()
    if "needs_layout_passes" in pltpu.CompilerParams.__dataclass_fields__:
        cp = dataclasses.replace(cp, needs_layout_passes=False)
    kern = pl.kernel(
        body,
        out_type=jax.ShapeDtypeStruct((batch, row_elems), jnp.float32),
        mesh=mesh,
        compiler_params=cp,
        scratch_types=[
            pltpu.VMEM((1, row_elems), jnp.float32),
            pltpu.VMEM((_GRP, seq), jnp.int32),
            pltpu.VMEM((1, row_elems), jnp.float32),
            pltpu.SemaphoreType.DMA,
        ],
    )
    return kern(inputs, emb_flat)


def _tc_body(inp_e_ref, inp_o_ref, emb_ref, out_ref, bcast_ref, dirty_ref, sem):
    i = pl.program_id(0)
    n = pl.num_programs(0)
    slot = jax.lax.rem(i, 2)
    bb, lp = inp_e_ref.shape
    d2 = emb_ref.shape[1]
    d = d2 // 2
    sub = bb // _K

    @pl.when(i == 0)
    def _build_bcast():
        bcast_ref[...] = jnp.broadcast_to(emb_ref[...][None, :, :], (bb, lp, d2))

    def _issue(src_ref, step, k, slt):
        return pltpu.make_async_copy(
            src_ref.at[pl.ds(k * sub, sub)],
            out_ref.at[pl.ds(step * bb + k * sub, sub)],
            sem.at[slt, k],
        )

    @pl.when(i >= 2)
    def _wait_prev():
        for k in range(_K):
            _issue(bcast_ref, i - 2, k, slot).wait()

    clean = jnp.logical_and(
        jnp.all(inp_e_ref[...] != PADDING_TOKEN),
        jnp.all(inp_o_ref[...] != PADDING_TOKEN),
    )

    @pl.when(clean)
    def _fast():
        for k in range(_K):
            _issue(bcast_ref, i, k, slot).start()

    @pl.when(jnp.logical_not(clean))
    def _masked():
        m_e = (inp_e_ref[...] != PADDING_TOKEN).astype(jnp.float32)[:, :, None]
        m_o = (inp_o_ref[...] != PADDING_TOKEN).astype(jnp.float32)[:, :, None]
        mask = jnp.concatenate(
            [jnp.broadcast_to(m_e, (bb, lp, d)), jnp.broadcast_to(m_o, (bb, lp, d))],
            axis=-1,
        )
        dirty_ref[slot] = mask * emb_ref[...][None, :, :]
        for k in range(_K):
            _issue(dirty_ref.at[slot], i, k, slot).start()

    @pl.when(i == n - 1)
    def _drain():
        for k in range(_K):
            _issue(bcast_ref, i - 1, k, 1 - slot).wait()
            _issue(bcast_ref, i, k, slot).wait()


def _tc_kernel_call(inputs, embedding_matrix):
    batch, seq = inputs.shape
    dim = embedding_matrix.shape[1]
    lp = seq // 2
    inputs_e = inputs[:, 0::2]
    inputs_o = inputs[:, 1::2]
    emb2 = embedding_matrix.reshape(lp, 2 * dim)
    grid = (batch // _BB,)
    return pl.pallas_call(
        _tc_body,
        grid=grid,
        in_specs=[
            pl.BlockSpec((_BB, lp), lambda i: (i, 0)),
            pl.BlockSpec((_BB, lp), lambda i: (i, 0)),
            pl.BlockSpec((lp, 2 * dim), lambda i: (0, 0)),
        ],
        out_specs=pl.BlockSpec(memory_space=pl.ANY),
        out_shape=jax.ShapeDtypeStruct((batch, lp, 2 * dim), jnp.float32),
        scratch_shapes=[
            pltpu.VMEM((_BB, lp, 2 * dim), jnp.float32),
            pltpu.VMEM((2, _BB, lp, 2 * dim), jnp.float32),
            pltpu.SemaphoreType.DMA((2, _K)),
        ],
    )(inputs_e, inputs_o, emb2)


def kernel(inputs, embedding_matrix):
    if inputs.shape[1] > MAX_CONTEXT:
        inputs = inputs[:, -MAX_CONTEXT:]
    batch, seq = inputs.shape
    dim = embedding_matrix.shape[1]
    row_elems = seq * dim
    emb_flat = embedding_matrix.reshape(1, row_elems)
    sc_rows = min(_SC_ROWS, batch)
    sc_out = _sc_kernel_call(inputs[:sc_rows], emb_flat, sc_rows, seq, row_elems)
    sc_out = sc_out.reshape(sc_rows, seq, dim)
    if sc_rows == batch:
        return sc_out
    tc_out = _tc_kernel_call(inputs[sc_rows:], embedding_matrix)
    tc_out = tc_out.reshape(batch - sc_rows, seq, dim)
    return jnp.concatenate([sc_out, tc_out], axis=0)


# speedup vs baseline: 1.4943x; 1.4943x over previous
import dataclasses

import jax
import jax.numpy as jnp
from jax import lax
from jax.experimental import pallas as pl
from jax.experimental.pallas import tpu as pltpu
from jax.experimental.pallas import tpu_sc as plsc

MAX_CONTEXT = 200
PADDING_TOKEN = 0

_NC = 2
_NS = 16
_NW = _NC * _NS
_GRP = 16

_CHUNK_OFFS = tuple(range(0, 192, 16)) + (184,)


def _sc_kernel_call(inputs, emb_flat, batch, seq, row_elems):
    rpw = batch // _NW
    ngrp = rpw // _GRP
    mesh = plsc.VectorSubcoreMesh(core_axis_name="c", subcore_axis_name="s")

    def body(inp_hbm, emb_hbm, out_hbm, emb_v, inp_v, dirty_v, sem, sem_i):
        c = lax.axis_index("c")
        s = lax.axis_index("s")
        wid = s * _NC + c
        base = wid * rpw
        pltpu.sync_copy(emb_hbm, emb_v)

        def _inp_dma(g, slot):
            return pltpu.make_async_copy(
                inp_hbm.at[pl.ds(base + g * _GRP, _GRP)],
                inp_v.at[slot],
                sem_i.at[slot],
            )

        def _drain(count):
            def _w(_, x):
                pltpu.make_async_copy(emb_v, out_hbm.at[pl.ds(base, 1)], sem).wait()
                return x

            lax.fori_loop(0, count, _w, 0)

        def _process_group(g, slot, prev_fired):
            row0 = base + g * _GRP
            fired = jnp.int32(0)
            for r in range(_GRP):
                ok = None
                for off in _CHUNK_OFFS:
                    nz = inp_v[slot, r, pl.ds(off, 16)] != PADDING_TOKEN
                    ok = nz if ok is None else jnp.logical_and(ok, nz)
                clean = jnp.all(ok)
                fired = fired + clean.astype(jnp.int32)

                @pl.when(clean)
                def _fast():
                    pltpu.make_async_copy(
                        emb_v, out_hbm.at[pl.ds(row0 + r, 1)], sem
                    ).start()

                @pl.when(jnp.logical_not(clean))
                def _masked():
                    lane = jnp.arange(16, dtype=jnp.int32)
                    zero16 = jnp.zeros((16,), jnp.int32)

                    @pl.loop(0, row_elems // 16)
                    def _chunk(ch):
                        col = ch * 16
                        tok = col // 64
                        tok16 = jnp.broadcast_to(tok, (16,))
                        s16 = jnp.full((16,), slot, jnp.int32)
                        r16 = jnp.full((16,), r, jnp.int32)
                        tokval = plsc.load_gather(inp_v, [s16, r16, tok16])
                        m = (tokval != PADDING_TOKEN).astype(jnp.float32)
                        cols = col + lane
                        ev = plsc.load_gather(emb_v, [jnp.zeros((16,), jnp.int32), cols])
                        plsc.store_scatter(dirty_v, [jnp.zeros((16,), jnp.int32), cols], ev * m)

                    pltpu.sync_copy(dirty_v, out_hbm.at[pl.ds(row0 + r, 1)])

            _drain(prev_fired)
            return fired

        _inp_dma(0, 0).start()

        def _pair(h, prev_fired):
            g0 = h * 2
            _inp_dma(g0, 0).wait()
            _inp_dma(g0 + 1, 1).start()
            f0 = _process_group(g0, 0, prev_fired)
            _inp_dma(g0 + 1, 1).wait()

            @pl.when(h + 1 < ngrp // 2)
            def _prefetch():
                _inp_dma(g0 + 2, 0).start()

            return _process_group(g0 + 1, 1, f0)

        last = lax.fori_loop(0, ngrp // 2, _pair, jnp.int32(0))
        _drain(last)

    cp = pltpu.CompilerParams()
    if "needs_layout_passes" in pltpu.CompilerParams.__dataclass_fields__:
        cp = dataclasses.replace(cp, needs_layout_passes=False)
    kern = pl.kernel(
        body,
        out_type=jax.ShapeDtypeStruct((batch, row_elems), jnp.float32),
        mesh=mesh,
        compiler_params=cp,
        scratch_types=[
            pltpu.VMEM((1, row_elems), jnp.float32),
            pltpu.VMEM((2, _GRP, seq), jnp.int32),
            pltpu.VMEM((1, row_elems), jnp.float32),
            pltpu.SemaphoreType.DMA,
            pltpu.SemaphoreType.DMA((2,)),
        ],
    )
    return kern(inputs, emb_flat)


def kernel(inputs, embedding_matrix):
    if inputs.shape[1] > MAX_CONTEXT:
        inputs = inputs[:, -MAX_CONTEXT:]
    batch, seq = inputs.shape
    dim = embedding_matrix.shape[1]
    row_elems = seq * dim
    emb_flat = embedding_matrix.reshape(1, row_elems)
    out2 = _sc_kernel_call(inputs, emb_flat, batch, seq, row_elems)
    return out2.reshape(batch, seq, dim)
